# Initial kernel scaffold; baseline (speedup 1.0000x reference)
import jax
import jax.numpy as jnp
from jax.experimental import pallas as pl


def kernel(seq_cat_data, seq_cont_data):
    raise NotImplementedError("write your pallas kernel here")


# speedup vs baseline: 1.5358x; 1.5358x over previous
import numpy as np
import jax
import jax.numpy as jnp
from jax import lax
from jax.experimental import pallas as pl
from jax.experimental.pallas import tpu as pltpu

L = 200
N_CAT = 26
N_CONT = 13
VOCAB = 100000
_ROT_A = (13, 15, 26, 6)
_ROT_B = (17, 29, 16, 24)


def _tf_host(k1, k2, x0, x1):
    M = 0xFFFFFFFF

    def rotl(v, r):
        return ((v << r) | (v >> (32 - r))) & M

    ks = (k1, k2, (k1 ^ k2 ^ 0x1BD11BDA) & M)
    x0 = (x0 + ks[0]) & M
    x1 = (x1 + ks[1]) & M
    for i in range(5):
        for r in (_ROT_A if i % 2 == 0 else _ROT_B):
            x0 = (x0 + x1) & M
            x1 = rotl(x1, r)
            x1 ^= x0
        x0 = (x0 + ks[(i + 1) % 3]) & M
        x1 = (x1 + ks[(i + 2) % 3] + i + 1) & M
    return x0, x1


_SUBS = [_tf_host(0, 42, 0, i) for i in range(5)]
_K_KEEP, _K_SUB, _K_MASK, _K_GAU, _K_SHUF = _SUBS
_K_RAND = _tf_host(_K_SUB[0], _K_SUB[1], 0, 1)


def _i32c(v):
    return jnp.int32(np.uint32(v).astype(np.int32))


def _rotl_v(x, r):
    return lax.shift_left(x, jnp.int32(r)) | lax.shift_right_logical(
        x, jnp.int32(32 - r))


def _tf_bits(key, x1):
    k1, k2 = key
    ks = (_i32c(k1), _i32c(k2), _i32c((k1 ^ k2 ^ 0x1BD11BDA) & 0xFFFFFFFF))
    x0 = jnp.full(x1.shape, ks[0], dtype=jnp.int32)
    x1 = x1 + ks[1]
    for i in range(5):
        for r in (_ROT_A if i % 2 == 0 else _ROT_B):
            x0 = x0 + x1
            x1 = _rotl_v(x1, r)
            x1 = x1 ^ x0
        x0 = x0 + ks[(i + 1) % 3]
        x1 = x1 + (ks[(i + 2) % 3] + jnp.int32(i + 1))
    return x0 ^ x1


def _bits_to_unit_f32(bits):
    m = lax.shift_right_logical(bits, jnp.int32(9)) | _i32c(0x3F800000)
    return lax.bitcast_convert_type(m, jnp.float32) - jnp.float32(1.0)


def _umod_vocab(bits):
    xf = bits.astype(jnp.float32) + jnp.where(
        bits < 0, jnp.float32(4294967296.0), jnp.float32(0.0))
    q = (xf * jnp.float32(1.0 / VOCAB)).astype(jnp.int32)
    r = bits - q * jnp.int32(VOCAB)
    r = r + jnp.where(r < 0, jnp.int32(VOCAB), jnp.int32(0))
    r = r + jnp.where(r < 0, jnp.int32(VOCAB), jnp.int32(0))
    r = r - jnp.where(r >= VOCAB, jnp.int32(VOCAB), jnp.int32(0))
    return r


def _erfinv_f32(u):
    w = -jnp.log1p(-u * u)
    lt = w < jnp.float32(5.0)
    w1 = w - jnp.float32(2.5)
    w2 = jnp.sqrt(jnp.maximum(w, jnp.float32(5.0))) - jnp.float32(3.0)
    c1 = (2.81022636e-08, 3.43273939e-07, -3.5233877e-06, -4.39150654e-06,
          0.00021858087, -0.00125372503, -0.00417768164, 0.246640727,
          1.50140941)
    c2 = (-0.000200214257, 0.000100950558, 0.00134934322, -0.00367342844,
          0.00573950773, -0.0076224613, 0.00943887047, 1.00167406,
          2.83297682)
    p1 = jnp.float32(c1[0])
    for c in c1[1:]:
        p1 = p1 * w1 + jnp.float32(c)
    p2 = jnp.float32(c2[0])
    for c in c2[1:]:
        p2 = p2 * w2 + jnp.float32(c)
    return jnp.where(lt, p1, p2) * u


def _expand(x, rep, width):
    r = x.shape[0]
    return jnp.broadcast_to(x[:, :, None], (r, L, rep)).reshape(r, width)


def _noise_kernel(cat_ref, cont_ref, cat_out_ref, cont_out_ref):
    rblk = cat_ref.shape[0]
    wc = L * N_CAT
    wf = L * N_CONT
    row0 = pl.program_id(0) * rblk

    def flat_idx(width):
        i = lax.broadcasted_iota(jnp.int32, (rblk, width), 0)
        j = lax.broadcasted_iota(jnp.int32, (rblk, width), 1)
        return (row0 + i) * width + j

    idx_c = flat_idx(wc)
    keep_u = _bits_to_unit_f32(_tf_bits(_K_KEEP, idx_c))
    rand_tok = _umod_vocab(_tf_bits(_K_RAND, idx_c))
    cat = jnp.where(keep_u > jnp.float32(0.1), cat_ref[...], rand_tok)

    idx_f = flat_idx(wf)
    gmask_u = _bits_to_unit_f32(_tf_bits(_K_MASK, idx_f))
    gmask = (gmask_u < jnp.float32(0.1)).astype(jnp.float32)
    gu = _bits_to_unit_f32(_tf_bits(_K_GAU, idx_f))
    lo = jnp.float32(np.nextafter(np.float32(-1.0), np.float32(0.0)))
    un = jnp.maximum(lo, gu * jnp.float32(2.0) + lo)
    z = jnp.float32(1.4142135381698608) * _erfinv_f32(un)
    cont = cont_ref[...] + z * gmask

    li = lax.broadcasted_iota(jnp.int32, (rblk, L), 1)
    inc_u = _bits_to_unit_f32(_tf_bits(_K_SHUF, (row0 + lax.broadcasted_iota(
        jnp.int32, (rblk, L), 0)) * L + li))
    keyv = li.astype(jnp.float32) + jnp.float32(2.0) * inc_u

    for rep, width, data, out_ref in ((N_CAT, wc, cat, cat_out_ref),
                                      (N_CONT, wf, cont, cont_out_ref)):
        ke = _expand(keyv, rep, width)
        je = lax.broadcasted_iota(jnp.int32, (rblk, width), 1)
        take_next = (jnp.roll(ke, -rep, axis=1) < ke) & (je < width - rep)
        take_prev = (ke < jnp.roll(ke, rep, axis=1)) & (je >= rep)
        shifted = jnp.where(take_next, jnp.roll(data, -rep, axis=1),
                            jnp.roll(data, rep, axis=1))
        out_ref[...] = jnp.where(take_next | take_prev, shifted, data)


def kernel(seq_cat_data, seq_cont_data):
    b = seq_cat_data.shape[0]
    rblk = 8
    cat2 = seq_cat_data.reshape(b, L * N_CAT)
    cont2 = seq_cont_data.reshape(b, L * N_CONT)
    cat_o, cont_o = pl.pallas_call(
        _noise_kernel,
        grid=(b // rblk,),
        in_specs=[
            pl.BlockSpec((rblk, L * N_CAT), lambda i: (i, 0)),
            pl.BlockSpec((rblk, L * N_CONT), lambda i: (i, 0)),
        ],
        out_specs=[
            pl.BlockSpec((rblk, L * N_CAT), lambda i: (i, 0)),
            pl.BlockSpec((rblk, L * N_CONT), lambda i: (i, 0)),
        ],
        out_shape=[
            jax.ShapeDtypeStruct((b, L * N_CAT), seq_cat_data.dtype),
            jax.ShapeDtypeStruct((b, L * N_CONT), seq_cont_data.dtype),
        ],
        compiler_params=pltpu.CompilerParams(
            dimension_semantics=("arbitrary",)),
    )(cat2, cont2)
    return (cat_o.reshape(b, L, N_CAT), cont_o.reshape(b, L, N_CONT))


# speedup vs baseline: 1.5359x; 1.0001x over previous
import numpy as np
import jax
import jax.numpy as jnp
from jax import lax
from jax.experimental import pallas as pl
from jax.experimental.pallas import tpu as pltpu

L = 200
N_CAT = 26
N_CONT = 13
VOCAB = 100000
_ROT_A = (13, 15, 26, 6)
_ROT_B = (17, 29, 16, 24)


def _tf_host(k1, k2, x0, x1):
    M = 0xFFFFFFFF

    def rotl(v, r):
        return ((v << r) | (v >> (32 - r))) & M

    ks = (k1, k2, (k1 ^ k2 ^ 0x1BD11BDA) & M)
    x0 = (x0 + ks[0]) & M
    x1 = (x1 + ks[1]) & M
    for i in range(5):
        for r in (_ROT_A if i % 2 == 0 else _ROT_B):
            x0 = (x0 + x1) & M
            x1 = rotl(x1, r)
            x1 ^= x0
        x0 = (x0 + ks[(i + 1) % 3]) & M
        x1 = (x1 + ks[(i + 2) % 3] + i + 1) & M
    return x0, x1


_SUBS = [_tf_host(0, 42, 0, i) for i in range(5)]
_K_KEEP, _K_SUB, _K_MASK, _K_GAU, _K_SHUF = _SUBS
_K_RAND = _tf_host(_K_SUB[0], _K_SUB[1], 0, 1)


def _i32c(v):
    return jnp.int32(np.uint32(v).astype(np.int32))


def _rotl_v(x, r):
    return lax.shift_left(x, jnp.int32(r)) | lax.shift_right_logical(
        x, jnp.int32(32 - r))


def _tf_bits(key, x1):
    k1, k2 = key
    ks = (_i32c(k1), _i32c(k2), _i32c((k1 ^ k2 ^ 0x1BD11BDA) & 0xFFFFFFFF))
    x0 = jnp.full(x1.shape, ks[0], dtype=jnp.int32)
    x1 = x1 + ks[1]
    for i in range(5):
        for r in (_ROT_A if i % 2 == 0 else _ROT_B):
            x0 = x0 + x1
            x1 = _rotl_v(x1, r)
            x1 = x1 ^ x0
        x0 = x0 + ks[(i + 1) % 3]
        x1 = x1 + (ks[(i + 2) % 3] + jnp.int32(i + 1))
    return x0 ^ x1


def _bits_to_unit_f32(bits):
    m = lax.shift_right_logical(bits, jnp.int32(9)) | _i32c(0x3F800000)
    return lax.bitcast_convert_type(m, jnp.float32) - jnp.float32(1.0)


def _umod_vocab(bits):
    xf = bits.astype(jnp.float32) + jnp.where(
        bits < 0, jnp.float32(4294967296.0), jnp.float32(0.0))
    q = (xf * jnp.float32(1.0 / VOCAB)).astype(jnp.int32)
    r = bits - q * jnp.int32(VOCAB)
    r = r + jnp.where(r < 0, jnp.int32(VOCAB), jnp.int32(0))
    r = r + jnp.where(r < 0, jnp.int32(VOCAB), jnp.int32(0))
    r = r - jnp.where(r >= VOCAB, jnp.int32(VOCAB), jnp.int32(0))
    return r


def _erfinv_f32(u):
    w = -jnp.log1p(-u * u)
    lt = w < jnp.float32(5.0)
    w1 = w - jnp.float32(2.5)
    w2 = jnp.sqrt(jnp.maximum(w, jnp.float32(5.0))) - jnp.float32(3.0)
    c1 = (2.81022636e-08, 3.43273939e-07, -3.5233877e-06, -4.39150654e-06,
          0.00021858087, -0.00125372503, -0.00417768164, 0.246640727,
          1.50140941)
    c2 = (-0.000200214257, 0.000100950558, 0.00134934322, -0.00367342844,
          0.00573950773, -0.0076224613, 0.00943887047, 1.00167406,
          2.83297682)
    p1 = jnp.float32(c1[0])
    for c in c1[1:]:
        p1 = p1 * w1 + jnp.float32(c)
    p2 = jnp.float32(c2[0])
    for c in c2[1:]:
        p2 = p2 * w2 + jnp.float32(c)
    return jnp.where(lt, p1, p2) * u


def _expand(x, rep, width):
    r = x.shape[0]
    return jnp.broadcast_to(x[:, :, None], (r, L, rep)).reshape(r, width)


def _noise_kernel(cat_ref, cont_ref, cat_out_ref, cont_out_ref):
    rblk = cat_ref.shape[0]
    wc = L * N_CAT
    wf = L * N_CONT
    row0 = pl.program_id(0) * rblk

    def flat_idx(width):
        i = lax.broadcasted_iota(jnp.int32, (rblk, width), 0)
        j = lax.broadcasted_iota(jnp.int32, (rblk, width), 1)
        return (row0 + i) * width + j

    idx_c = flat_idx(wc)
    keep_u = _bits_to_unit_f32(_tf_bits(_K_KEEP, idx_c))
    rand_tok = _umod_vocab(_tf_bits(_K_RAND, idx_c))
    cat = jnp.where(keep_u > jnp.float32(0.1), cat_ref[...], rand_tok)

    idx_f = flat_idx(wf)
    gmask_u = _bits_to_unit_f32(_tf_bits(_K_MASK, idx_f))
    gmask = (gmask_u < jnp.float32(0.1)).astype(jnp.float32)
    gu = _bits_to_unit_f32(_tf_bits(_K_GAU, idx_f))
    lo = jnp.float32(np.nextafter(np.float32(-1.0), np.float32(0.0)))
    un = jnp.maximum(lo, gu * jnp.float32(2.0) + lo)
    z = jnp.float32(1.4142135381698608) * _erfinv_f32(un)
    cont = cont_ref[...] + z * gmask

    li = lax.broadcasted_iota(jnp.int32, (rblk, L), 1)
    inc_u = _bits_to_unit_f32(_tf_bits(_K_SHUF, (row0 + lax.broadcasted_iota(
        jnp.int32, (rblk, L), 0)) * L + li))
    keyv = li.astype(jnp.float32) + jnp.float32(2.0) * inc_u

    for rep, width, data, out_ref in ((N_CAT, wc, cat, cat_out_ref),
                                      (N_CONT, wf, cont, cont_out_ref)):
        ke = _expand(keyv, rep, width)
        je = lax.broadcasted_iota(jnp.int32, (rblk, width), 1)
        take_next = (jnp.roll(ke, -rep, axis=1) < ke) & (je < width - rep)
        take_prev = (ke < jnp.roll(ke, rep, axis=1)) & (je >= rep)
        shifted = jnp.where(take_next, jnp.roll(data, -rep, axis=1),
                            jnp.roll(data, rep, axis=1))
        out_ref[...] = jnp.where(take_next | take_prev, shifted, data)


def kernel(seq_cat_data, seq_cont_data):
    b = seq_cat_data.shape[0]
    rblk = 8
    cat2 = seq_cat_data.reshape(b, L * N_CAT)
    cont2 = seq_cont_data.reshape(b, L * N_CONT)
    cat_o, cont_o = pl.pallas_call(
        _noise_kernel,
        grid=(b // rblk,),
        in_specs=[
            pl.BlockSpec((rblk, L * N_CAT), lambda i: (i, 0)),
            pl.BlockSpec((rblk, L * N_CONT), lambda i: (i, 0)),
        ],
        out_specs=[
            pl.BlockSpec((rblk, L * N_CAT), lambda i: (i, 0)),
            pl.BlockSpec((rblk, L * N_CONT), lambda i: (i, 0)),
        ],
        out_shape=[
            jax.ShapeDtypeStruct((b, L * N_CAT), seq_cat_data.dtype),
            jax.ShapeDtypeStruct((b, L * N_CONT), seq_cont_data.dtype),
        ],
        compiler_params=pltpu.CompilerParams(
            dimension_semantics=("parallel",)),
    )(cat2, cont2)
    return (cat_o.reshape(b, L, N_CAT), cont_o.reshape(b, L, N_CONT))


# speedup vs baseline: 1.6191x; 1.0542x over previous
import numpy as np
import jax
import jax.numpy as jnp
from jax import lax
from jax.experimental import pallas as pl
from jax.experimental.pallas import tpu as pltpu

L = 200
N_CAT = 26
N_CONT = 13
VOCAB = 100000
_ROT_A = (13, 15, 26, 6)
_ROT_B = (17, 29, 16, 24)


def _tf_host(k1, k2, x0, x1):
    M = 0xFFFFFFFF

    def rotl(v, r):
        return ((v << r) | (v >> (32 - r))) & M

    ks = (k1, k2, (k1 ^ k2 ^ 0x1BD11BDA) & M)
    x0 = (x0 + ks[0]) & M
    x1 = (x1 + ks[1]) & M
    for i in range(5):
        for r in (_ROT_A if i % 2 == 0 else _ROT_B):
            x0 = (x0 + x1) & M
            x1 = rotl(x1, r)
            x1 ^= x0
        x0 = (x0 + ks[(i + 1) % 3]) & M
        x1 = (x1 + ks[(i + 2) % 3] + i + 1) & M
    return x0, x1


_SUBS = [_tf_host(0, 42, 0, i) for i in range(5)]
_K_KEEP, _K_SUB, _K_MASK, _K_GAU, _K_SHUF = _SUBS
_K_RAND = _tf_host(_K_SUB[0], _K_SUB[1], 0, 1)


def _i32c(v):
    return jnp.int32(np.uint32(v).astype(np.int32))


def _rotl_v(x, r):
    return lax.shift_left(x, jnp.int32(r)) | lax.shift_right_logical(
        x, jnp.int32(32 - r))


def _tf_bits(key, x1):
    k1, k2 = key
    ks = (_i32c(k1), _i32c(k2), _i32c((k1 ^ k2 ^ 0x1BD11BDA) & 0xFFFFFFFF))
    x0 = jnp.full(x1.shape, ks[0], dtype=jnp.int32)
    x1 = x1 + ks[1]
    for i in range(5):
        for r in (_ROT_A if i % 2 == 0 else _ROT_B):
            x0 = x0 + x1
            x1 = _rotl_v(x1, r)
            x1 = x1 ^ x0
        x0 = x0 + ks[(i + 1) % 3]
        x1 = x1 + (ks[(i + 2) % 3] + jnp.int32(i + 1))
    return x0 ^ x1


def _bits_to_unit_f32(bits):
    m = lax.shift_right_logical(bits, jnp.int32(9)) | _i32c(0x3F800000)
    return lax.bitcast_convert_type(m, jnp.float32) - jnp.float32(1.0)


def _umod_vocab(bits):
    xf = bits.astype(jnp.float32) + jnp.where(
        bits < 0, jnp.float32(4294967296.0), jnp.float32(0.0))
    q = (xf * jnp.float32(1.0 / VOCAB)).astype(jnp.int32)
    r = bits - q * jnp.int32(VOCAB)
    r = r + jnp.where(r < 0, jnp.int32(VOCAB), jnp.int32(0))
    r = r - jnp.where(r >= VOCAB, jnp.int32(VOCAB), jnp.int32(0))
    return r


def _erfinv_f32(u):
    w = -jnp.log1p(-u * u)
    lt = w < jnp.float32(5.0)
    w1 = w - jnp.float32(2.5)
    w2 = jnp.sqrt(jnp.maximum(w, jnp.float32(5.0))) - jnp.float32(3.0)
    c1 = (2.81022636e-08, 3.43273939e-07, -3.5233877e-06, -4.39150654e-06,
          0.00021858087, -0.00125372503, -0.00417768164, 0.246640727,
          1.50140941)
    c2 = (-0.000200214257, 0.000100950558, 0.00134934322, -0.00367342844,
          0.00573950773, -0.0076224613, 0.00943887047, 1.00167406,
          2.83297682)
    p1 = jnp.float32(c1[0])
    for c in c1[1:]:
        p1 = p1 * w1 + jnp.float32(c)
    p2 = jnp.float32(c2[0])
    for c in c2[1:]:
        p2 = p2 * w2 + jnp.float32(c)
    return jnp.where(lt, p1, p2) * u


_M_TENTH = 0x0CCCCD


def _noise_kernel(cat_ref, cont_ref, cat_out_ref, cont_out_ref, e26_ref,
                  e13_ref, ic_ref, if_ref, il_ref):
    rblk = cat_ref.shape[0]
    wc = L * N_CAT
    wf = L * N_CONT
    row0 = pl.program_id(0) * rblk

    @pl.when(pl.program_id(0) == 0)
    def _():
        for rep, width, ref in ((N_CAT, wc, e26_ref), (N_CONT, wf, e13_ref)):
            lv = lax.broadcasted_iota(jnp.int32, (L, width), 0)
            jv = lax.broadcasted_iota(jnp.int32, (L, width), 1)
            t = jv - jnp.int32(rep) * lv
            one = (t >= 0) & (t < rep)
            ref[...] = one.astype(jnp.bfloat16)
        for width, ref in ((wc, ic_ref), (wf, if_ref), (L, il_ref)):
            iv = lax.broadcasted_iota(jnp.int32, (rblk, width), 0)
            jv = lax.broadcasted_iota(jnp.int32, (rblk, width), 1)
            ref[...] = iv * jnp.int32(width) + jv

    idx_c = ic_ref[...] + row0 * jnp.int32(wc)
    keep_m = lax.shift_right_logical(_tf_bits(_K_KEEP, idx_c), jnp.int32(9))
    rand_tok = _umod_vocab(_tf_bits(_K_RAND, idx_c))
    cat = jnp.where(keep_m > jnp.int32(_M_TENTH), cat_ref[...], rand_tok)

    idx_f = if_ref[...] + row0 * jnp.int32(wf)
    gmask_m = lax.shift_right_logical(_tf_bits(_K_MASK, idx_f), jnp.int32(9))
    gmask = (gmask_m < jnp.int32(_M_TENTH)).astype(jnp.float32)
    gu = _bits_to_unit_f32(_tf_bits(_K_GAU, idx_f))
    lo = jnp.float32(np.nextafter(np.float32(-1.0), np.float32(0.0)))
    un = jnp.maximum(lo, gu * jnp.float32(2.0) + lo)
    z = jnp.float32(1.4142135381698608) * _erfinv_f32(un)
    cont = cont_ref[...] + z * gmask

    li = lax.broadcasted_iota(jnp.int32, (rblk, L), 1)
    inc_u = _bits_to_unit_f32(
        _tf_bits(_K_SHUF, il_ref[...] + row0 * jnp.int32(L)))
    keyv = li.astype(jnp.float32) + jnp.float32(2.0) * inc_u
    tn = (jnp.roll(keyv, -1, axis=1) < keyv) & (li < jnp.int32(L - 1))
    tp = (keyv < jnp.roll(keyv, 1, axis=1)) & (li > jnp.int32(0))
    tn_bf = tn.astype(jnp.bfloat16)
    tp_bf = tp.astype(jnp.bfloat16)
    dn = (((1,), (0,)), ((), ()))

    for rep, data, out_ref, e_ref in ((N_CAT, cat, cat_out_ref, e26_ref),
                                      (N_CONT, cont, cont_out_ref, e13_ref)):
        tn_w = lax.dot_general(tn_bf, e_ref[...], dn,
                               preferred_element_type=jnp.float32)
        tp_w = lax.dot_general(tp_bf, e_ref[...], dn,
                               preferred_element_type=jnp.float32)
        half = jnp.float32(0.5)
        shifted = jnp.where(tn_w > half, jnp.roll(data, -rep, axis=1),
                            jnp.roll(data, rep, axis=1))
        out_ref[...] = jnp.where((tn_w + tp_w) > half, shifted, data)


def kernel(seq_cat_data, seq_cont_data):
    b = seq_cat_data.shape[0]
    rblk = 8
    cat2 = seq_cat_data.reshape(b, L * N_CAT)
    cont2 = seq_cont_data.reshape(b, L * N_CONT)
    cat_o, cont_o = pl.pallas_call(
        _noise_kernel,
        grid=(b // rblk,),
        in_specs=[
            pl.BlockSpec((rblk, L * N_CAT), lambda i: (i, 0)),
            pl.BlockSpec((rblk, L * N_CONT), lambda i: (i, 0)),
        ],
        out_specs=[
            pl.BlockSpec((rblk, L * N_CAT), lambda i: (i, 0)),
            pl.BlockSpec((rblk, L * N_CONT), lambda i: (i, 0)),
        ],
        out_shape=[
            jax.ShapeDtypeStruct((b, L * N_CAT), seq_cat_data.dtype),
            jax.ShapeDtypeStruct((b, L * N_CONT), seq_cont_data.dtype),
        ],
        scratch_shapes=[
            pltpu.VMEM((L, L * N_CAT), jnp.bfloat16),
            pltpu.VMEM((L, L * N_CONT), jnp.bfloat16),
            pltpu.VMEM((rblk, L * N_CAT), jnp.int32),
            pltpu.VMEM((rblk, L * N_CONT), jnp.int32),
            pltpu.VMEM((rblk, L), jnp.int32),
        ],
        compiler_params=pltpu.CompilerParams(
            dimension_semantics=("arbitrary",)),
    )(cat2, cont2)
    return (cat_o.reshape(b, L, N_CAT), cont_o.reshape(b, L, N_CONT))


# speedup vs baseline: 1.8485x; 1.1417x over previous
import numpy as np
import jax
import jax.numpy as jnp
from jax import lax
from jax.experimental import pallas as pl
from jax.experimental.pallas import tpu as pltpu

L = 200
N_CAT = 26
N_CONT = 13
VOCAB = 100000
_ROT_A = (13, 15, 26, 6)
_ROT_B = (17, 29, 16, 24)


def _tf_host(k1, k2, x0, x1):
    M = 0xFFFFFFFF

    def rotl(v, r):
        return ((v << r) | (v >> (32 - r))) & M

    ks = (k1, k2, (k1 ^ k2 ^ 0x1BD11BDA) & M)
    x0 = (x0 + ks[0]) & M
    x1 = (x1 + ks[1]) & M
    for i in range(5):
        for r in (_ROT_A if i % 2 == 0 else _ROT_B):
            x0 = (x0 + x1) & M
            x1 = rotl(x1, r)
            x1 ^= x0
        x0 = (x0 + ks[(i + 1) % 3]) & M
        x1 = (x1 + ks[(i + 2) % 3] + i + 1) & M
    return x0, x1


_SUBS = [_tf_host(0, 42, 0, i) for i in range(5)]
_K_KEEP, _K_SUB, _K_MASK, _K_GAU, _K_SHUF = _SUBS
_K_RAND = _tf_host(_K_SUB[0], _K_SUB[1], 0, 1)


def _i32c(v):
    return jnp.int32(np.uint32(v).astype(np.int32))


def _rotl_v(x, r):
    return lax.shift_left(x, jnp.int32(r)) | lax.shift_right_logical(
        x, jnp.int32(32 - r))


def _tf_bits(key, x1):
    k1, k2 = key
    ks = (_i32c(k1), _i32c(k2), _i32c((k1 ^ k2 ^ 0x1BD11BDA) & 0xFFFFFFFF))
    x0 = jnp.full(x1.shape, ks[0], dtype=jnp.int32)
    x1 = x1 + ks[1]
    for i in range(5):
        for r in (_ROT_A if i % 2 == 0 else _ROT_B):
            x0 = x0 + x1
            x1 = _rotl_v(x1, r)
            x1 = x1 ^ x0
        x0 = x0 + ks[(i + 1) % 3]
        x1 = x1 + (ks[(i + 2) % 3] + jnp.int32(i + 1))
    return x0 ^ x1


def _bits_to_unit_f32(bits):
    m = lax.shift_right_logical(bits, jnp.int32(9)) | _i32c(0x3F800000)
    return lax.bitcast_convert_type(m, jnp.float32) - jnp.float32(1.0)


def _umod_vocab(bits):
    xf = bits.astype(jnp.float32) + jnp.where(
        bits < 0, jnp.float32(4294967296.0), jnp.float32(0.0))
    q = (xf * jnp.float32(1.0 / VOCAB)).astype(jnp.int32)
    r = bits - q * jnp.int32(VOCAB)
    r = r + jnp.where(r < 0, jnp.int32(VOCAB), jnp.int32(0))
    r = r - jnp.where(r >= VOCAB, jnp.int32(VOCAB), jnp.int32(0))
    return r


def _erfinv_f32(u):
    w = -jnp.log1p(-u * u)
    lt = w < jnp.float32(5.0)
    w1 = w - jnp.float32(2.5)
    w2 = jnp.sqrt(jnp.maximum(w, jnp.float32(5.0))) - jnp.float32(3.0)
    c1 = (2.81022636e-08, 3.43273939e-07, -3.5233877e-06, -4.39150654e-06,
          0.00021858087, -0.00125372503, -0.00417768164, 0.246640727,
          1.50140941)
    c2 = (-0.000200214257, 0.000100950558, 0.00134934322, -0.00367342844,
          0.00573950773, -0.0076224613, 0.00943887047, 1.00167406,
          2.83297682)
    p1 = jnp.float32(c1[0])
    for c in c1[1:]:
        p1 = p1 * w1 + jnp.float32(c)
    p2 = jnp.float32(c2[0])
    for c in c2[1:]:
        p2 = p2 * w2 + jnp.float32(c)
    return jnp.where(lt, p1, p2) * u


_M_GT = 0x0CCCCC
_M_LT = 0x0CCCCD


def _noise_kernel(cat_ref, cont_ref, cat_out_ref, cont_out_ref, e26_ref,
                  e13_ref, ic_ref, if_ref, il_ref):
    rblk = cat_ref.shape[0]
    wc = L * N_CAT
    wf = L * N_CONT
    row0 = pl.program_id(0) * rblk

    @pl.when(pl.program_id(0) == 0)
    def _():
        for rep, width, ref in ((N_CAT, wc, e26_ref), (N_CONT, wf, e13_ref)):
            lv = lax.broadcasted_iota(jnp.int32, (L, width), 0)
            jv = lax.broadcasted_iota(jnp.int32, (L, width), 1)
            t = jv - jnp.int32(rep) * lv
            one = (t >= 0) & (t < rep)
            ref[...] = one.astype(jnp.bfloat16)
        for width, ref in ((wc, ic_ref), (wf, if_ref), (L, il_ref)):
            iv = lax.broadcasted_iota(jnp.int32, (rblk, width), 0)
            jv = lax.broadcasted_iota(jnp.int32, (rblk, width), 1)
            ref[...] = iv * jnp.int32(width) + jv

    idx_c = ic_ref[...] + row0 * jnp.int32(wc)
    keep_m = lax.shift_right_logical(_tf_bits(_K_KEEP, idx_c), jnp.int32(9))
    rand_tok = _umod_vocab(_tf_bits(_K_RAND, idx_c))
    cat = jnp.where(keep_m > jnp.int32(_M_GT), cat_ref[...], rand_tok)

    idx_f = if_ref[...] + row0 * jnp.int32(wf)
    gmask_m = lax.shift_right_logical(_tf_bits(_K_MASK, idx_f), jnp.int32(9))
    gmask = (gmask_m < jnp.int32(_M_LT)).astype(jnp.float32)
    gu = _bits_to_unit_f32(_tf_bits(_K_GAU, idx_f))
    lo = jnp.float32(np.nextafter(np.float32(-1.0), np.float32(0.0)))
    un = jnp.maximum(lo, gu * jnp.float32(2.0) + lo)
    z = jnp.float32(1.4142135381698608) * _erfinv_f32(un)
    cont = cont_ref[...] + z * gmask

    li = lax.broadcasted_iota(jnp.int32, (rblk, L), 1)
    inc_u = _bits_to_unit_f32(
        _tf_bits(_K_SHUF, il_ref[...] + row0 * jnp.int32(L)))
    keyv = li.astype(jnp.float32) + jnp.float32(2.0) * inc_u
    tn = (jnp.roll(keyv, -1, axis=1) < keyv) & (li < jnp.int32(L - 1))
    tp = (keyv < jnp.roll(keyv, 1, axis=1)) & (li > jnp.int32(0))
    tn_bf = tn.astype(jnp.bfloat16)
    tp_bf = tp.astype(jnp.bfloat16)
    dn = (((1,), (0,)), ((), ()))

    for rep, data, out_ref, e_ref in ((N_CAT, cat, cat_out_ref, e26_ref),
                                      (N_CONT, cont, cont_out_ref, e13_ref)):
        tn_w = lax.dot_general(tn_bf, e_ref[...], dn,
                               preferred_element_type=jnp.float32)
        tp_w = lax.dot_general(tp_bf, e_ref[...], dn,
                               preferred_element_type=jnp.float32)
        half = jnp.float32(0.5)
        shifted = jnp.where(tn_w > half, jnp.roll(data, -rep, axis=1),
                            jnp.roll(data, rep, axis=1))
        out_ref[...] = jnp.where((tn_w + tp_w) > half, shifted, data)


def kernel(seq_cat_data, seq_cont_data):
    b = seq_cat_data.shape[0]
    rblk = 32
    cat2 = seq_cat_data.reshape(b, L * N_CAT)
    cont2 = seq_cont_data.reshape(b, L * N_CONT)
    cat_o, cont_o = pl.pallas_call(
        _noise_kernel,
        grid=(b // rblk,),
        in_specs=[
            pl.BlockSpec((rblk, L * N_CAT), lambda i: (i, 0)),
            pl.BlockSpec((rblk, L * N_CONT), lambda i: (i, 0)),
        ],
        out_specs=[
            pl.BlockSpec((rblk, L * N_CAT), lambda i: (i, 0)),
            pl.BlockSpec((rblk, L * N_CONT), lambda i: (i, 0)),
        ],
        out_shape=[
            jax.ShapeDtypeStruct((b, L * N_CAT), seq_cat_data.dtype),
            jax.ShapeDtypeStruct((b, L * N_CONT), seq_cont_data.dtype),
        ],
        scratch_shapes=[
            pltpu.VMEM((L, L * N_CAT), jnp.bfloat16),
            pltpu.VMEM((L, L * N_CONT), jnp.bfloat16),
            pltpu.VMEM((rblk, L * N_CAT), jnp.int32),
            pltpu.VMEM((rblk, L * N_CONT), jnp.int32),
            pltpu.VMEM((rblk, L), jnp.int32),
        ],
        compiler_params=pltpu.CompilerParams(
            dimension_semantics=("arbitrary",)),
    )(cat2, cont2)
    return (cat_o.reshape(b, L, N_CAT), cont_o.reshape(b, L, N_CONT))


# speedup vs baseline: 1.9110x; 1.0338x over previous
import numpy as np
import jax
import jax.numpy as jnp
from jax import lax
from jax.experimental import pallas as pl
from jax.experimental.pallas import tpu as pltpu

L = 200
N_CAT = 26
N_CONT = 13
VOCAB = 100000
_ROT_A = (13, 15, 26, 6)
_ROT_B = (17, 29, 16, 24)


def _tf_host(k1, k2, x0, x1):
    M = 0xFFFFFFFF

    def rotl(v, r):
        return ((v << r) | (v >> (32 - r))) & M

    ks = (k1, k2, (k1 ^ k2 ^ 0x1BD11BDA) & M)
    x0 = (x0 + ks[0]) & M
    x1 = (x1 + ks[1]) & M
    for i in range(5):
        for r in (_ROT_A if i % 2 == 0 else _ROT_B):
            x0 = (x0 + x1) & M
            x1 = rotl(x1, r)
            x1 ^= x0
        x0 = (x0 + ks[(i + 1) % 3]) & M
        x1 = (x1 + ks[(i + 2) % 3] + i + 1) & M
    return x0, x1


_SUBS = [_tf_host(0, 42, 0, i) for i in range(5)]
_K_KEEP, _K_SUB, _K_MASK, _K_GAU, _K_SHUF = _SUBS
_K_RAND = _tf_host(_K_SUB[0], _K_SUB[1], 0, 1)


def _i32c(v):
    return jnp.int32(np.uint32(v).astype(np.int32))


def _rotl_v(x, r):
    return lax.shift_left(x, jnp.int32(r)) | lax.shift_right_logical(
        x, jnp.int32(32 - r))


def _tf_bits(key, x1):
    k1, k2 = key
    ks = (_i32c(k1), _i32c(k2), _i32c((k1 ^ k2 ^ 0x1BD11BDA) & 0xFFFFFFFF))
    x0 = jnp.full(x1.shape, ks[0], dtype=jnp.int32)
    x1 = x1 + ks[1]
    for i in range(5):
        for r in (_ROT_A if i % 2 == 0 else _ROT_B):
            x0 = x0 + x1
            x1 = _rotl_v(x1, r)
            x1 = x1 ^ x0
        x0 = x0 + ks[(i + 1) % 3]
        x1 = x1 + (ks[(i + 2) % 3] + jnp.int32(i + 1))
    return x0 ^ x1


def _bits_to_unit_f32(bits):
    m = lax.shift_right_logical(bits, jnp.int32(9)) | _i32c(0x3F800000)
    return lax.bitcast_convert_type(m, jnp.float32) - jnp.float32(1.0)


def _umod_vocab(bits):
    xf = bits.astype(jnp.float32) + jnp.where(
        bits < 0, jnp.float32(4294967296.0), jnp.float32(0.0))
    q = (xf * jnp.float32(1.0 / VOCAB)).astype(jnp.int32)
    r = bits - q * jnp.int32(VOCAB)
    r = r + jnp.where(r < 0, jnp.int32(VOCAB), jnp.int32(0))
    r = r - jnp.where(r >= VOCAB, jnp.int32(VOCAB), jnp.int32(0))
    return r


def _erfinv_f32(u):
    w = -jnp.log1p(-u * u)
    lt = w < jnp.float32(5.0)
    w1 = w - jnp.float32(2.5)
    w2 = jnp.sqrt(jnp.maximum(w, jnp.float32(5.0))) - jnp.float32(3.0)
    c1 = (2.81022636e-08, 3.43273939e-07, -3.5233877e-06, -4.39150654e-06,
          0.00021858087, -0.00125372503, -0.00417768164, 0.246640727,
          1.50140941)
    c2 = (-0.000200214257, 0.000100950558, 0.00134934322, -0.00367342844,
          0.00573950773, -0.0076224613, 0.00943887047, 1.00167406,
          2.83297682)
    p1 = jnp.float32(c1[0])
    for c in c1[1:]:
        p1 = p1 * w1 + jnp.float32(c)
    p2 = jnp.float32(c2[0])
    for c in c2[1:]:
        p2 = p2 * w2 + jnp.float32(c)
    return jnp.where(lt, p1, p2) * u


_M_GT = 0x0CCCCC
_M_LT = 0x0CCCCD


def _noise_kernel(cat_ref, cont_ref, cat_out_ref, cont_out_ref, e26_ref,
                  e13_ref, ic_ref, if_ref, il_ref):
    rblk = cat_ref.shape[0]
    wc = L * N_CAT
    wf = L * N_CONT
    row0 = pl.program_id(0) * rblk

    @pl.when(pl.program_id(0) == 0)
    def _():
        for rep, width, ref in ((N_CAT, wc, e26_ref), (N_CONT, wf, e13_ref)):
            lv = lax.broadcasted_iota(jnp.int32, (L, width), 0)
            jv = lax.broadcasted_iota(jnp.int32, (L, width), 1)
            t = jv - jnp.int32(rep) * lv
            one = (t >= 0) & (t < rep)
            ref[...] = one.astype(jnp.bfloat16)
        for width, ref in ((wc, ic_ref), (wf, if_ref), (L, il_ref)):
            iv = lax.broadcasted_iota(jnp.int32, (rblk, width), 0)
            jv = lax.broadcasted_iota(jnp.int32, (rblk, width), 1)
            ref[...] = iv * jnp.int32(width) + jv

    idx_c = ic_ref[...] + row0 * jnp.int32(wc)
    keep_m = lax.shift_right_logical(_tf_bits(_K_KEEP, idx_c), jnp.int32(9))
    rand_tok = _umod_vocab(_tf_bits(_K_RAND, idx_c))
    cat = jnp.where(keep_m > jnp.int32(_M_GT), cat_ref[...], rand_tok)

    idx_f = if_ref[...] + row0 * jnp.int32(wf)
    gmask_m = lax.shift_right_logical(_tf_bits(_K_MASK, idx_f), jnp.int32(9))
    gmask = (gmask_m < jnp.int32(_M_LT)).astype(jnp.float32)
    gu = _bits_to_unit_f32(_tf_bits(_K_GAU, idx_f))
    lo = jnp.float32(np.nextafter(np.float32(-1.0), np.float32(0.0)))
    un = jnp.maximum(lo, gu * jnp.float32(2.0) + lo)
    z = jnp.float32(1.4142135381698608) * _erfinv_f32(un)
    cont = cont_ref[...] + z * gmask

    li = lax.broadcasted_iota(jnp.int32, (rblk, L), 1)
    inc_u = _bits_to_unit_f32(
        _tf_bits(_K_SHUF, il_ref[...] + row0 * jnp.int32(L)))
    keyv = li.astype(jnp.float32) + jnp.float32(2.0) * inc_u
    tn = (jnp.roll(keyv, -1, axis=1) < keyv) & (li < jnp.int32(L - 1))
    tp = (keyv < jnp.roll(keyv, 1, axis=1)) & (li > jnp.int32(0))
    tn_bf = tn.astype(jnp.bfloat16)
    tp_bf = tp.astype(jnp.bfloat16)
    dn = (((1,), (0,)), ((), ()))

    for rep, data, out_ref, e_ref in ((N_CAT, cat, cat_out_ref, e26_ref),
                                      (N_CONT, cont, cont_out_ref, e13_ref)):
        tn_w = lax.dot_general(tn_bf, e_ref[...], dn,
                               preferred_element_type=jnp.float32)
        tp_w = lax.dot_general(tp_bf, e_ref[...], dn,
                               preferred_element_type=jnp.float32)
        half = jnp.float32(0.5)
        shifted = jnp.where(tn_w > half, jnp.roll(data, -rep, axis=1),
                            jnp.roll(data, rep, axis=1))
        out_ref[...] = jnp.where((tn_w + tp_w) > half, shifted, data)


def kernel(seq_cat_data, seq_cont_data):
    b = seq_cat_data.shape[0]
    rblk = 64
    cat2 = seq_cat_data.reshape(b, L * N_CAT)
    cont2 = seq_cont_data.reshape(b, L * N_CONT)
    cat_o, cont_o = pl.pallas_call(
        _noise_kernel,
        grid=(b // rblk,),
        in_specs=[
            pl.BlockSpec((rblk, L * N_CAT), lambda i: (i, 0)),
            pl.BlockSpec((rblk, L * N_CONT), lambda i: (i, 0)),
        ],
        out_specs=[
            pl.BlockSpec((rblk, L * N_CAT), lambda i: (i, 0)),
            pl.BlockSpec((rblk, L * N_CONT), lambda i: (i, 0)),
        ],
        out_shape=[
            jax.ShapeDtypeStruct((b, L * N_CAT), seq_cat_data.dtype),
            jax.ShapeDtypeStruct((b, L * N_CONT), seq_cont_data.dtype),
        ],
        scratch_shapes=[
            pltpu.VMEM((L, L * N_CAT), jnp.bfloat16),
            pltpu.VMEM((L, L * N_CONT), jnp.bfloat16),
            pltpu.VMEM((rblk, L * N_CAT), jnp.int32),
            pltpu.VMEM((rblk, L * N_CONT), jnp.int32),
            pltpu.VMEM((rblk, L), jnp.int32),
        ],
        compiler_params=pltpu.CompilerParams(
            dimension_semantics=("arbitrary",)),
    )(cat2, cont2)
    return (cat_o.reshape(b, L, N_CAT), cont_o.reshape(b, L, N_CONT))
